# Initial kernel scaffold; baseline (speedup 1.0000x reference)
import jax
import jax.numpy as jnp
from jax.experimental import pallas as pl


def kernel(x, edge_index, W1l, b1l, W1r, W2l, b2l, W2r, W3l, b3l, W3r):
    raise NotImplementedError("write your pallas kernel here")


# speedup vs baseline: 3.0983x; 3.0983x over previous
import jax
import jax.numpy as jnp
from jax import lax
from jax.experimental import pallas as pl
from jax.experimental.pallas import tpu as pltpu
from jax.experimental.pallas import tpu_sc as plsc

N = 10000
E = 320000
D = 128
H = 256

NC = 2
NS = 16
CK = 128
CH = 80
EPAD = NC * NS * CH * CK
NP = 10240
RPT = NP // NS
DST_PAD = NP - 1


def _make_sc_agg(W):
  def body(src_hbm, dst_hbm, tbl_hbm, out_hbm, agg_sh, src_v, dst_v, rows_v,
           sem):
    c = lax.axis_index("c")
    s = lax.axis_index("s")

    nzw = W // 16

    def zstep(i, _):
      r = i // nzw
      q = i % nzw
      rows_v[r, pl.ds(q * 16, 16)] = jnp.zeros((16,), jnp.float32)
      return 0

    lax.fori_loop(0, CK * nzw, zstep, 0)

    def zcopy(k, _):
      pltpu.sync_copy(rows_v, agg_sh.at[pl.ds(s * RPT + k * CK, CK)])
      return 0

    lax.fori_loop(0, RPT // CK, zcopy, 0)

    plsc.subcore_barrier()

    pltpu.sync_copy(src_hbm.at[c, s], src_v)
    pltpu.sync_copy(dst_hbm.at[c, s], dst_v)

    def step(j, _):
      pltpu.async_copy(tbl_hbm.at[src_v.at[j]], rows_v, sem).wait()
      pltpu.sync_copy(rows_v, agg_sh.at[dst_v.at[j]], add=True)
      return 0

    lax.fori_loop(0, CH, step, 0)

    plsc.subcore_barrier()

    pltpu.sync_copy(agg_sh.at[pl.ds(s * RPT, RPT)],
                    out_hbm.at[c, pl.ds(s * RPT, RPT)])

  mesh = plsc.VectorSubcoreMesh(core_axis_name="c", subcore_axis_name="s",
                                num_cores=NC, num_subcores=NS)
  return pl.kernel(
      body,
      out_type=jax.ShapeDtypeStruct((NC, NP, W), jnp.float32),
      mesh=mesh,
      scratch_types=[
          pltpu.VMEM_SHARED((NP, W), jnp.float32),
          pltpu.VMEM((CH, CK), jnp.int32),
          pltpu.VMEM((CH, CK), jnp.int32),
          pltpu.VMEM((CK, W), jnp.float32),
          pltpu.SemaphoreType.DMA,
      ],
  )


_sc_agg128 = _make_sc_agg(128)


def _sc_cnt_body(dst_hbm, cnt_hbm, cnt_sh, dst_v, ones_v):
  c = lax.axis_index("c")
  s = lax.axis_index("s")

  def zstep(i, _):
    r = i // 8
    q = i % 8
    ones_v[r, pl.ds(q * 16, 16)] = jnp.zeros((16,), jnp.float32)
    return 0

  lax.fori_loop(0, CK * 8, zstep, 0)

  def zcopy(k, _):
    pltpu.sync_copy(ones_v, cnt_sh.at[pl.ds(s * RPT + k * CK, CK)])
    return 0

  lax.fori_loop(0, RPT // CK, zcopy, 0)

  def ostep(i, _):
    r = i // 8
    q = i % 8
    ones_v[r, pl.ds(q * 16, 16)] = jnp.ones((16,), jnp.float32)
    return 0

  lax.fori_loop(0, CK * 8, ostep, 0)

  plsc.subcore_barrier()

  pltpu.sync_copy(dst_hbm.at[c, s], dst_v)

  def step(j, _):
    pltpu.sync_copy(ones_v, cnt_sh.at[dst_v.at[j]], add=True)
    return 0

  lax.fori_loop(0, CH, step, 0)

  plsc.subcore_barrier()

  pltpu.sync_copy(cnt_sh.at[pl.ds(s * RPT, RPT)],
                  cnt_hbm.at[c, pl.ds(s * RPT, RPT)])


_sc_cnt = pl.kernel(
    _sc_cnt_body,
    out_type=jax.ShapeDtypeStruct((NC, NP, D), jnp.float32),
    mesh=plsc.VectorSubcoreMesh(core_axis_name="c", subcore_axis_name="s",
                                num_cores=NC, num_subcores=NS),
    scratch_types=[
        pltpu.VMEM_SHARED((NP, D), jnp.float32),
        pltpu.VMEM((CH, CK), jnp.int32),
        pltpu.VMEM((CK, D), jnp.float32),
    ],
)


def _inv_deg(cnt_ref):
  cnt = cnt_ref[0, :, 0:1] + cnt_ref[1, :, 0:1]
  return 1.0 / jnp.maximum(cnt, 1.0)


def _l2n(t):
  nrm = jnp.sqrt(jnp.sum(t * t, axis=1, keepdims=True))
  return t / jnp.maximum(nrm, 1e-12)


def _layer1_body(agg_ref, cnt_ref, x_ref, w1l_ref, b1l_ref, w1r_ref, w2l_ref,
                 h1_ref, p2_ref):
  mean = (agg_ref[0] + agg_ref[1]) * _inv_deg(cnt_ref)
  t = (jnp.dot(mean, w1l_ref[...], preferred_element_type=jnp.float32)
       + b1l_ref[...]
       + jnp.dot(x_ref[...], w1r_ref[...], preferred_element_type=jnp.float32))
  h = jnp.maximum(_l2n(t), 0.0)
  h1_ref[...] = h
  p2_ref[...] = jnp.dot(h, w2l_ref[...], preferred_element_type=jnp.float32)


def _layer2_body(agg_ref, cnt_ref, h1_ref, b2l_ref, w2r_ref, h2_ref):
  t = ((agg_ref[0] + agg_ref[1]) * _inv_deg(cnt_ref)
       + b2l_ref[...]
       + jnp.dot(h1_ref[...], w2r_ref[...],
                 preferred_element_type=jnp.float32))
  h2_ref[...] = jnp.maximum(_l2n(t), 0.0)


def _layer3_body(agg_ref, cnt_ref, h2_ref, b3l_ref, w3l_ref, w3r_ref,
                 out_ref):
  mean = (agg_ref[0] + agg_ref[1]) * _inv_deg(cnt_ref)
  t = (jnp.dot(mean, w3l_ref[...], preferred_element_type=jnp.float32)
       + b3l_ref[...]
       + jnp.dot(h2_ref[...], w3r_ref[...],
                 preferred_element_type=jnp.float32))
  t = _l2n(t)
  mask = lax.broadcasted_iota(jnp.int32, t.shape, 1) < 2
  tm = jnp.where(mask, t, -1e30)
  m = jnp.max(tm, axis=1, keepdims=True)
  lse = m + jnp.log(jnp.sum(jnp.where(mask, jnp.exp(tm - m), 0.0),
                            axis=1, keepdims=True))
  out_ref[...] = tm - lse


_RB = 1000


def _tc_layer1(agg, cnt, x, w1l, b1l, w1r, w2l):
  return pl.pallas_call(
      _layer1_body,
      grid=(N // _RB,),
      in_specs=[
          pl.BlockSpec((NC, _RB, D), lambda i: (0, i, 0)),
          pl.BlockSpec((NC, _RB, D), lambda i: (0, i, 0)),
          pl.BlockSpec((_RB, D), lambda i: (i, 0)),
          pl.BlockSpec((D, H), lambda i: (0, 0)),
          pl.BlockSpec((1, H), lambda i: (0, 0)),
          pl.BlockSpec((D, H), lambda i: (0, 0)),
          pl.BlockSpec((H, D), lambda i: (0, 0)),
      ],
      out_specs=[
          pl.BlockSpec((_RB, H), lambda i: (i, 0)),
          pl.BlockSpec((_RB, D), lambda i: (i, 0)),
      ],
      out_shape=[
          jax.ShapeDtypeStruct((N, H), jnp.float32),
          jax.ShapeDtypeStruct((N, D), jnp.float32),
      ],
  )(agg, cnt, x, w1l, b1l.reshape(1, H), w1r, w2l)


def _tc_layer2(agg, cnt, h1, b2l, w2r):
  return pl.pallas_call(
      _layer2_body,
      grid=(N // _RB,),
      in_specs=[
          pl.BlockSpec((NC, _RB, D), lambda i: (0, i, 0)),
          pl.BlockSpec((NC, _RB, D), lambda i: (0, i, 0)),
          pl.BlockSpec((_RB, H), lambda i: (i, 0)),
          pl.BlockSpec((1, D), lambda i: (0, 0)),
          pl.BlockSpec((H, D), lambda i: (0, 0)),
      ],
      out_specs=pl.BlockSpec((_RB, D), lambda i: (i, 0)),
      out_shape=jax.ShapeDtypeStruct((N, D), jnp.float32),
  )(agg, cnt, h1, b2l.reshape(1, D), w2r)


def _tc_layer3(agg, cnt, h2, b3l_pad, w3l_pad, w3r_pad):
  return pl.pallas_call(
      _layer3_body,
      grid=(N // _RB,),
      in_specs=[
          pl.BlockSpec((NC, _RB, D), lambda i: (0, i, 0)),
          pl.BlockSpec((NC, _RB, D), lambda i: (0, i, 0)),
          pl.BlockSpec((_RB, D), lambda i: (i, 0)),
          pl.BlockSpec((1, 16), lambda i: (0, 0)),
          pl.BlockSpec((D, 16), lambda i: (0, 0)),
          pl.BlockSpec((D, 16), lambda i: (0, 0)),
      ],
      out_specs=pl.BlockSpec((_RB, 16), lambda i: (i, 0)),
      out_shape=jax.ShapeDtypeStruct((N, 16), jnp.float32),
  )(agg, cnt, h2, b3l_pad.reshape(1, 16), w3l_pad, w3r_pad)


@jax.jit
def _run(x, edge_index, W1l, b1l, W1r, W2l, b2l, W2r, W3l, b3l, W3r):
  npad = EPAD - E
  src = jnp.concatenate(
      [edge_index[0], jnp.zeros((npad,), jnp.int32)]).reshape(NC, NS, CH, CK)
  dst = jnp.concatenate(
      [edge_index[1], jnp.full((npad,), DST_PAD, jnp.int32)]
  ).reshape(NC, NS, CH, CK)

  cnt = _sc_cnt(dst)
  agg1 = _sc_agg128(src, dst, x)
  h1, p2 = _tc_layer1(agg1, cnt, x, W1l, b1l, W1r, W2l)

  agg2 = _sc_agg128(src, dst, p2)
  h2 = _tc_layer2(agg2, cnt, h1, b2l, W2r)

  agg3 = _sc_agg128(src, dst, h2)
  w3l_pad = jnp.pad(W3l, ((0, 0), (0, 14)))
  b3l_pad = jnp.pad(b3l, (0, 14))
  w3r_pad = jnp.pad(W3r, ((0, 0), (0, 14)))
  out16 = _tc_layer3(agg3, cnt, h2, b3l_pad, w3l_pad, w3r_pad)
  return out16[:, :2]


def kernel(x, edge_index, W1l, b1l, W1r, W2l, b2l, W2r, W3l, b3l, W3r):
  return _run(x, edge_index, W1l, b1l, W1r, W2l, b2l, W2r, W3l, b3l, W3r)


# speedup vs baseline: 3.4198x; 1.1037x over previous
import jax
import jax.numpy as jnp
from jax import lax
from jax.experimental import pallas as pl
from jax.experimental.pallas import tpu as pltpu
from jax.experimental.pallas import tpu_sc as plsc

N = 10000
E = 320000
D = 128
H = 256

NC = 2
NS = 16
CK = 128
CH = 80
CHB = 40
EPAD = NC * NS * CH * CK
NP = 10240
RPT = NP // NS
DST_PAD = NP - 1


def _make_sc_agg(W):
  def body(src_hbm, dst_hbm, tbl_hbm, out_hbm, agg_sh, src_v, dst_v, rows0_v,
           rows1_v, sem0, sem1):
    c = lax.axis_index("c")
    s = lax.axis_index("s")

    nzw = W // 16

    def zstep(i, _):
      r = i // nzw
      q = i % nzw
      rows0_v[r, pl.ds(q * 16, 16)] = jnp.zeros((16,), jnp.float32)
      return 0

    lax.fori_loop(0, CK * nzw, zstep, 0)

    def zcopy(k, _):
      pltpu.sync_copy(rows0_v, agg_sh.at[pl.ds(s * RPT + k * CK, CK)])
      return 0

    lax.fori_loop(0, RPT // CK, zcopy, 0)

    plsc.subcore_barrier()

    for h in range(CH // CHB):
      pltpu.sync_copy(src_hbm.at[c, s, pl.ds(h * CHB, CHB)], src_v)
      pltpu.sync_copy(dst_hbm.at[c, s, pl.ds(h * CHB, CHB)], dst_v)

      pltpu.async_copy(tbl_hbm.at[src_v.at[0]], rows0_v, sem0)

      def step(j, _):
        @pl.when(j % 2 == 1)
        def _():
          pltpu.async_copy(tbl_hbm.at[src_v.at[j]], rows1_v, sem1)
          pltpu.make_async_copy(tbl_hbm.at[src_v.at[j - 1]], rows0_v,
                                sem0).wait()
          pltpu.sync_copy(rows0_v, agg_sh.at[dst_v.at[j - 1]], add=True)

        @pl.when(j % 2 == 0)
        def _():
          pltpu.async_copy(tbl_hbm.at[src_v.at[j]], rows0_v, sem0)
          pltpu.make_async_copy(tbl_hbm.at[src_v.at[j - 1]], rows1_v,
                                sem1).wait()
          pltpu.sync_copy(rows1_v, agg_sh.at[dst_v.at[j - 1]], add=True)

        return 0

      lax.fori_loop(1, CHB, step, 0)

      pltpu.make_async_copy(tbl_hbm.at[src_v.at[CHB - 1]], rows1_v,
                            sem1).wait()
      pltpu.sync_copy(rows1_v, agg_sh.at[dst_v.at[CHB - 1]], add=True)

    plsc.subcore_barrier()

    pltpu.sync_copy(agg_sh.at[pl.ds(s * RPT, RPT)],
                    out_hbm.at[c, pl.ds(s * RPT, RPT)])

  mesh = plsc.VectorSubcoreMesh(core_axis_name="c", subcore_axis_name="s",
                                num_cores=NC, num_subcores=NS)
  return pl.kernel(
      body,
      out_type=jax.ShapeDtypeStruct((NC, NP, W), jnp.float32),
      mesh=mesh,
      scratch_types=[
          pltpu.VMEM_SHARED((NP, W), jnp.float32),
          pltpu.VMEM((CHB, CK), jnp.int32),
          pltpu.VMEM((CHB, CK), jnp.int32),
          pltpu.VMEM((CK, W), jnp.float32),
          pltpu.VMEM((CK, W), jnp.float32),
          pltpu.SemaphoreType.DMA,
          pltpu.SemaphoreType.DMA,
      ],
  )


_sc_agg128 = _make_sc_agg(128)


def _sc_cnt_body(dst_hbm, cnt_hbm, cnt_sh, dst_v, ones_v):
  c = lax.axis_index("c")
  s = lax.axis_index("s")

  def zstep(i, _):
    r = i // 8
    q = i % 8
    ones_v[r, pl.ds(q * 16, 16)] = jnp.zeros((16,), jnp.float32)
    return 0

  lax.fori_loop(0, CK * 8, zstep, 0)

  def zcopy(k, _):
    pltpu.sync_copy(ones_v, cnt_sh.at[pl.ds(s * RPT + k * CK, CK)])
    return 0

  lax.fori_loop(0, RPT // CK, zcopy, 0)

  def ostep(i, _):
    r = i // 8
    q = i % 8
    ones_v[r, pl.ds(q * 16, 16)] = jnp.ones((16,), jnp.float32)
    return 0

  lax.fori_loop(0, CK * 8, ostep, 0)

  plsc.subcore_barrier()

  pltpu.sync_copy(dst_hbm.at[c, s], dst_v)

  def step(j, _):
    pltpu.sync_copy(ones_v, cnt_sh.at[dst_v.at[j]], add=True)
    return 0

  lax.fori_loop(0, CH, step, 0)

  plsc.subcore_barrier()

  pltpu.sync_copy(cnt_sh.at[pl.ds(s * RPT, RPT)],
                  cnt_hbm.at[c, pl.ds(s * RPT, RPT)])


_sc_cnt = pl.kernel(
    _sc_cnt_body,
    out_type=jax.ShapeDtypeStruct((NC, NP, D), jnp.float32),
    mesh=plsc.VectorSubcoreMesh(core_axis_name="c", subcore_axis_name="s",
                                num_cores=NC, num_subcores=NS),
    scratch_types=[
        pltpu.VMEM_SHARED((NP, D), jnp.float32),
        pltpu.VMEM((CH, CK), jnp.int32),
        pltpu.VMEM((CK, D), jnp.float32),
    ],
)


def _inv_deg(cnt_ref):
  cnt = cnt_ref[0, :, 0:1] + cnt_ref[1, :, 0:1]
  return 1.0 / jnp.maximum(cnt, 1.0)


def _l2n(t):
  nrm = jnp.sqrt(jnp.sum(t * t, axis=1, keepdims=True))
  return t / jnp.maximum(nrm, 1e-12)


def _layer1_body(agg_ref, cnt_ref, x_ref, w1l_ref, b1l_ref, w1r_ref, w2l_ref,
                 h1_ref, p2_ref):
  mean = (agg_ref[0] + agg_ref[1]) * _inv_deg(cnt_ref)
  t = (jnp.dot(mean, w1l_ref[...], preferred_element_type=jnp.float32)
       + b1l_ref[...]
       + jnp.dot(x_ref[...], w1r_ref[...], preferred_element_type=jnp.float32))
  h = jnp.maximum(_l2n(t), 0.0)
  h1_ref[...] = h
  p2_ref[...] = jnp.dot(h, w2l_ref[...], preferred_element_type=jnp.float32)


def _layer2_body(agg_ref, cnt_ref, h1_ref, b2l_ref, w2r_ref, h2_ref):
  t = ((agg_ref[0] + agg_ref[1]) * _inv_deg(cnt_ref)
       + b2l_ref[...]
       + jnp.dot(h1_ref[...], w2r_ref[...],
                 preferred_element_type=jnp.float32))
  h2_ref[...] = jnp.maximum(_l2n(t), 0.0)


def _layer3_body(agg_ref, cnt_ref, h2_ref, b3l_ref, w3l_ref, w3r_ref,
                 out_ref):
  mean = (agg_ref[0] + agg_ref[1]) * _inv_deg(cnt_ref)
  t = (jnp.dot(mean, w3l_ref[...], preferred_element_type=jnp.float32)
       + b3l_ref[...]
       + jnp.dot(h2_ref[...], w3r_ref[...],
                 preferred_element_type=jnp.float32))
  t = _l2n(t)
  mask = lax.broadcasted_iota(jnp.int32, t.shape, 1) < 2
  tm = jnp.where(mask, t, -1e30)
  m = jnp.max(tm, axis=1, keepdims=True)
  lse = m + jnp.log(jnp.sum(jnp.where(mask, jnp.exp(tm - m), 0.0),
                            axis=1, keepdims=True))
  out_ref[...] = tm - lse


_RB = 1000


def _tc_layer1(agg, cnt, x, w1l, b1l, w1r, w2l):
  return pl.pallas_call(
      _layer1_body,
      grid=(N // _RB,),
      in_specs=[
          pl.BlockSpec((NC, _RB, D), lambda i: (0, i, 0)),
          pl.BlockSpec((NC, _RB, D), lambda i: (0, i, 0)),
          pl.BlockSpec((_RB, D), lambda i: (i, 0)),
          pl.BlockSpec((D, H), lambda i: (0, 0)),
          pl.BlockSpec((1, H), lambda i: (0, 0)),
          pl.BlockSpec((D, H), lambda i: (0, 0)),
          pl.BlockSpec((H, D), lambda i: (0, 0)),
      ],
      out_specs=[
          pl.BlockSpec((_RB, H), lambda i: (i, 0)),
          pl.BlockSpec((_RB, D), lambda i: (i, 0)),
      ],
      out_shape=[
          jax.ShapeDtypeStruct((N, H), jnp.float32),
          jax.ShapeDtypeStruct((N, D), jnp.float32),
      ],
  )(agg, cnt, x, w1l, b1l.reshape(1, H), w1r, w2l)


def _tc_layer2(agg, cnt, h1, b2l, w2r):
  return pl.pallas_call(
      _layer2_body,
      grid=(N // _RB,),
      in_specs=[
          pl.BlockSpec((NC, _RB, D), lambda i: (0, i, 0)),
          pl.BlockSpec((NC, _RB, D), lambda i: (0, i, 0)),
          pl.BlockSpec((_RB, H), lambda i: (i, 0)),
          pl.BlockSpec((1, D), lambda i: (0, 0)),
          pl.BlockSpec((H, D), lambda i: (0, 0)),
      ],
      out_specs=pl.BlockSpec((_RB, D), lambda i: (i, 0)),
      out_shape=jax.ShapeDtypeStruct((N, D), jnp.float32),
  )(agg, cnt, h1, b2l.reshape(1, D), w2r)


def _tc_layer3(agg, cnt, h2, b3l_pad, w3l_pad, w3r_pad):
  return pl.pallas_call(
      _layer3_body,
      grid=(N // _RB,),
      in_specs=[
          pl.BlockSpec((NC, _RB, D), lambda i: (0, i, 0)),
          pl.BlockSpec((NC, _RB, D), lambda i: (0, i, 0)),
          pl.BlockSpec((_RB, D), lambda i: (i, 0)),
          pl.BlockSpec((1, 16), lambda i: (0, 0)),
          pl.BlockSpec((D, 16), lambda i: (0, 0)),
          pl.BlockSpec((D, 16), lambda i: (0, 0)),
      ],
      out_specs=pl.BlockSpec((_RB, 16), lambda i: (i, 0)),
      out_shape=jax.ShapeDtypeStruct((N, 16), jnp.float32),
  )(agg, cnt, h2, b3l_pad.reshape(1, 16), w3l_pad, w3r_pad)


@jax.jit
def _run(x, edge_index, W1l, b1l, W1r, W2l, b2l, W2r, W3l, b3l, W3r):
  npad = EPAD - E
  src = jnp.concatenate(
      [edge_index[0], jnp.zeros((npad,), jnp.int32)]).reshape(NC, NS, CH, CK)
  dst = jnp.concatenate(
      [edge_index[1], jnp.full((npad,), DST_PAD, jnp.int32)]
  ).reshape(NC, NS, CH, CK)

  cnt = _sc_cnt(dst)
  agg1 = _sc_agg128(src, dst, x)
  h1, p2 = _tc_layer1(agg1, cnt, x, W1l, b1l, W1r, W2l)

  agg2 = _sc_agg128(src, dst, p2)
  h2 = _tc_layer2(agg2, cnt, h1, b2l, W2r)

  agg3 = _sc_agg128(src, dst, h2)
  w3l_pad = jnp.pad(W3l, ((0, 0), (0, 14)))
  b3l_pad = jnp.pad(b3l, (0, 14))
  w3r_pad = jnp.pad(W3r, ((0, 0), (0, 14)))
  out16 = _tc_layer3(agg3, cnt, h2, b3l_pad, w3l_pad, w3r_pad)
  return out16[:, :2]


def kernel(x, edge_index, W1l, b1l, W1r, W2l, b2l, W2r, W3l, b3l, W3r):
  return _run(x, edge_index, W1l, b1l, W1r, W2l, b2l, W2r, W3l, b3l, W3r)
